# baseline (speedup 1.0000x reference)
import jax
import jax.numpy as jnp
from jax import lax
from jax.experimental import pallas as pl
from jax.experimental.pallas import tpu as pltpu
from jax.experimental.pallas import tpu_sc as plsc

ROWS = 128
N = 32768
L = 16
NC = 2
NS = 16
NW = NC * NS
RPW = ROWS // NW
CPS = 16
SC_ELEMS = CPS * L
NSUP = N // SC_ELEMS
MAX_NEWTON = 24
NEG = -3.0e38


def _butterfly_max(v, idx):
    for sh in (8, 4, 2, 1):
        v = jnp.maximum(v, v[jnp.bitwise_xor(idx, sh)])
    return v


def _butterfly_sum(v, idx):
    for sh in (8, 4, 2, 1):
        v = v + v[jnp.bitwise_xor(idx, sh)]
    return v


def _sc_body(x_hbm, out_hbm, buf, segmax, sclist):
    wid = lax.axis_index("s") * NC + lax.axis_index("c")
    idx16 = lax.iota(jnp.int32, L)

    def do_row(r, _):
        row = wid * RPW + r
        pltpu.sync_copy(x_hbm.at[row], buf)

        def sup_body(j, gmax):
            def ch_body(c, m):
                return jnp.maximum(m, buf[pl.ds(j * SC_ELEMS + c * L, L)])

            m = lax.fori_loop(0, CPS, ch_body,
                              jnp.full((L,), NEG, jnp.float32), unroll=CPS)
            segmax[pl.ds(j * L, L)] = m
            return jnp.maximum(gmax, m)

        gmax = lax.fori_loop(0, NSUP, sup_body,
                             jnp.full((L,), NEG, jnp.float32))
        tau0 = _butterfly_max(gmax, idx16)[0] - 1.0

        def filt_body(j, nsc):
            m = segmax[pl.ds(j * L, L)]
            smax = _butterfly_max(m, idx16)[0]
            cond = smax > tau0

            @pl.when(cond)
            def _():
                sclist[nsc] = j

            return nsc + jnp.where(cond, 1, 0)

        nsc = lax.fori_loop(0, NSUP, filt_body, jnp.int32(0))

        zv = jnp.zeros((L,), jnp.float32)
        onev = jnp.full((L,), 1.0, jnp.float32)

        def newton_it(t, carry):
            tau, changed = carry
            trip = jnp.where(changed == 1, nsc, 0)
            tv = jnp.full((L,), tau, jnp.float32)

            def sc_body(u, acc):
                sa, ka = acc
                base = sclist[u] * SC_ELEMS
                for c in range(CPS):
                    v = buf[pl.ds(base + c * L, L)]
                    msk = v > tv
                    sa = sa + jnp.where(msk, v, zv)
                    ka = ka + jnp.where(msk, onev, zv)
                return sa, ka

            sa, ka = lax.fori_loop(0, trip, sc_body, (zv, zv))
            sv = _butterfly_sum(sa, idx16)
            kv = jnp.maximum(_butterfly_sum(ka, idx16), onev)
            tau_new = ((sv - 1.0) / kv)[0]
            keep = changed == 1
            tau_out = jnp.where(keep, tau_new, tau)
            changed_out = jnp.where(
                jnp.logical_and(keep, tau_new != tau), 1, 0).astype(jnp.int32)
            return tau_out, changed_out

        tau, _ = lax.fori_loop(0, MAX_NEWTON, newton_it,
                               (tau0, jnp.int32(1)))

        tvo = jnp.full((L,), tau, jnp.float32)

        def ob(i, _):
            v = buf[pl.ds(i * L, L)]
            buf[pl.ds(i * L, L)] = jnp.maximum(v - tvo, zv)
            return 0

        lax.fori_loop(0, N // L, ob, 0, unroll=8)
        pltpu.sync_copy(buf, out_hbm.at[row])
        return 0

    lax.fori_loop(0, RPW, do_row, 0)


def kernel(input):
    mesh = plsc.VectorSubcoreMesh(
        core_axis_name="c", subcore_axis_name="s", num_cores=NC, num_subcores=NS)
    fn = pl.kernel(
        _sc_body,
        out_type=jax.ShapeDtypeStruct((ROWS, N), jnp.float32),
        mesh=mesh,
        scratch_types=[
            pltpu.VMEM((N,), jnp.float32),
            pltpu.VMEM((NSUP * L,), jnp.float32),
            pltpu.SMEM((NSUP,), jnp.int32),
        ],
    )
    return fn(input)


# speedup vs baseline: 1.1185x; 1.1185x over previous
import jax
import jax.numpy as jnp
from jax import lax
from jax.experimental import pallas as pl
from jax.experimental.pallas import tpu as pltpu
from jax.experimental.pallas import tpu_sc as plsc

ROWS = 128
N = 32768
L = 16
NC = 2
NS = 16
NW = NC * NS
RPW = ROWS // NW
CPS = 16
SC_ELEMS = CPS * L
NSUP = N // SC_ELEMS
MAX_NEWTON = 24
NEG = -3.0e38


def _butterfly_max(v, idx):
    for sh in (8, 4, 2, 1):
        v = jnp.maximum(v, v[jnp.bitwise_xor(idx, sh)])
    return v


def _butterfly_sum(v, idx):
    for sh in (8, 4, 2, 1):
        v = v + v[jnp.bitwise_xor(idx, sh)]
    return v


def _compute_row(buf, segmax, sclist, idx16):
    zv = jnp.zeros((L,), jnp.float32)
    onev = jnp.full((L,), 1.0, jnp.float32)

    def sup_body(j, gmax):
        def ch_body(c, m):
            return jnp.maximum(m, buf[pl.ds(j * SC_ELEMS + c * L, L)])

        m = lax.fori_loop(0, CPS, ch_body,
                          jnp.full((L,), NEG, jnp.float32), unroll=CPS)
        segmax[pl.ds(j * L, L)] = m
        return jnp.maximum(gmax, m)

    gmax = lax.fori_loop(0, NSUP, sup_body,
                         jnp.full((L,), NEG, jnp.float32))
    tau0 = _butterfly_max(gmax, idx16)[0] - 1.0

    def filt_body(j, nsc):
        m = segmax[pl.ds(j * L, L)]
        smax = _butterfly_max(m, idx16)[0]
        cond = smax > tau0

        @pl.when(cond)
        def _():
            sclist[nsc] = j

        return nsc + jnp.where(cond, 1, 0)

    nsc = lax.fori_loop(0, NSUP, filt_body, jnp.int32(0))

    def newton_it(t, carry):
        tau, changed = carry
        trip = jnp.where(changed == 1, nsc, 0)
        tv = jnp.full((L,), tau, jnp.float32)

        def sc_body(u, acc):
            sa, ka = acc
            base = sclist[u] * SC_ELEMS
            for c in range(CPS):
                v = buf[pl.ds(base + c * L, L)]
                msk = v > tv
                sa = sa + jnp.where(msk, v, zv)
                ka = ka + jnp.where(msk, onev, zv)
            return sa, ka

        sa, ka = lax.fori_loop(0, trip, sc_body, (zv, zv))
        sv = _butterfly_sum(sa, idx16)
        kv = jnp.maximum(_butterfly_sum(ka, idx16), onev)
        tau_new = ((sv - 1.0) / kv)[0]
        keep = changed == 1
        tau_out = jnp.where(keep, tau_new, tau)
        changed_out = jnp.where(
            jnp.logical_and(keep, tau_new != tau), 1, 0).astype(jnp.int32)
        return tau_out, changed_out

    tau, _ = lax.fori_loop(0, MAX_NEWTON, newton_it, (tau0, jnp.int32(1)))

    tvo = jnp.full((L,), tau, jnp.float32)

    def ob(i, _):
        v = buf[pl.ds(i * L, L)]
        buf[pl.ds(i * L, L)] = jnp.maximum(v - tvo, zv)
        return 0

    lax.fori_loop(0, N // L, ob, 0, unroll=16)


def _sc_body(x_hbm, out_hbm, b0, b1, b2, segmax, sclist,
             si0, si1, si2, si3, so0, so1, so2, so3):
    wid = lax.axis_index("s") * NC + lax.axis_index("c")
    idx16 = lax.iota(jnp.int32, L)
    r0 = wid * RPW

    bufs = (b0, b1, b2, b0)
    isems = (si0, si1, si2, si3)
    osems = (so0, so1, so2, so3)

    h_in = [None] * RPW
    h_out = [None] * RPW

    h_in[0] = pltpu.async_copy(x_hbm.at[r0 + 0], bufs[0], isems[0])
    h_in[1] = pltpu.async_copy(x_hbm.at[r0 + 1], bufs[1], isems[1])
    for r in range(RPW):
        h_in[r].wait()
        _compute_row(bufs[r], segmax, sclist, idx16)
        h_out[r] = pltpu.async_copy(bufs[r], out_hbm.at[r0 + r], osems[r])
        if r + 2 < RPW:
            if r + 2 == 3:
                h_out[0].wait()
            h_in[r + 2] = pltpu.async_copy(
                x_hbm.at[r0 + r + 2], bufs[r + 2], isems[r + 2])
    h_out[1].wait()
    h_out[2].wait()
    h_out[3].wait()


def kernel(input):
    mesh = plsc.VectorSubcoreMesh(
        core_axis_name="c", subcore_axis_name="s", num_cores=NC, num_subcores=NS)
    fn = pl.kernel(
        _sc_body,
        out_type=jax.ShapeDtypeStruct((ROWS, N), jnp.float32),
        mesh=mesh,
        scratch_types=[
            pltpu.VMEM((N,), jnp.float32),
            pltpu.VMEM((N,), jnp.float32),
            pltpu.VMEM((N,), jnp.float32),
            pltpu.VMEM((NSUP * L,), jnp.float32),
            pltpu.SMEM((NSUP,), jnp.int32),
            pltpu.SemaphoreType.DMA,
            pltpu.SemaphoreType.DMA,
            pltpu.SemaphoreType.DMA,
            pltpu.SemaphoreType.DMA,
            pltpu.SemaphoreType.DMA,
            pltpu.SemaphoreType.DMA,
            pltpu.SemaphoreType.DMA,
            pltpu.SemaphoreType.DMA,
        ],
    )
    return fn(input)


# speedup vs baseline: 1.2480x; 1.1157x over previous
import jax
import jax.numpy as jnp
from jax import lax
from jax.experimental import pallas as pl
from jax.experimental.pallas import tpu as pltpu
from jax.experimental.pallas import tpu_sc as plsc

ROWS = 128
N = 32768
L = 16
NC = 2
NS = 16
NW = NC * NS
RPW = ROWS // NW
CPS = 16
SC_ELEMS = CPS * L
NSUP = N // SC_ELEMS
MAX_NEWTON = 20
NEG = -3.0e38


def _bmax(v, idx):
    for sh in (8, 4, 2, 1):
        v = jnp.maximum(v, v[jnp.bitwise_xor(idx, sh)])
    return v


def _bsum(v, idx):
    for sh in (8, 4, 2, 1):
        v = v + v[jnp.bitwise_xor(idx, sh)]
    return v


def _newton_step(buf, sclist, nsc, tau, changed, idx16, zv, onev):
    trip = jnp.where(changed == 1, nsc, 0)
    tv = jnp.full((L,), tau, jnp.float32)

    def sc_body(u, acc):
        sa, ka = acc
        base = sclist[u] * SC_ELEMS
        for c in range(CPS):
            v = buf[pl.ds(base + c * L, L)]
            msk = v > tv
            sa = sa + jnp.where(msk, v, zv)
            ka = ka + jnp.where(msk, onev, zv)
        return sa, ka

    sa, ka = lax.fori_loop(0, trip, sc_body, (zv, zv))
    sv = _bsum(sa, idx16)
    kv = jnp.maximum(_bsum(ka, idx16), onev)
    tau_new = ((sv - 1.0) / kv)[0]
    keep = changed == 1
    tau_out = jnp.where(keep, tau_new, tau)
    changed_out = jnp.where(
        jnp.logical_and(keep, tau_new != tau), 1, 0).astype(jnp.int32)
    return tau_out, changed_out


def _compute_row(buf, segmax, sclist, idx16):
    zv = jnp.zeros((L,), jnp.float32)
    onev = jnp.full((L,), 1.0, jnp.float32)

    def sup_body(j, gmax):
        def ch_body(c, m):
            return jnp.maximum(m, buf[pl.ds(j * SC_ELEMS + c * L, L)])

        m = lax.fori_loop(0, CPS, ch_body,
                          jnp.full((L,), NEG, jnp.float32), unroll=CPS)
        segmax[pl.ds(j * L, L)] = m
        return jnp.maximum(gmax, m)

    gmax = lax.fori_loop(0, NSUP, sup_body,
                         jnp.full((L,), NEG, jnp.float32))
    tau0 = _bmax(gmax, idx16)[0] - 1.0

    def filt_body(j, nsc):
        m = segmax[pl.ds(j * L, L)]
        smax = _bmax(m, idx16)[0]
        cond = smax > tau0

        @pl.when(cond)
        def _():
            sclist[nsc] = j

        return nsc + jnp.where(cond, 1, 0)

    nsc = lax.fori_loop(0, NSUP, filt_body, jnp.int32(0))

    tau1, ch1 = _newton_step(buf, sclist, nsc, tau0, jnp.int32(1),
                             idx16, zv, onev)

    def rf_body(u, cnt):
        j = sclist[u]
        m = segmax[pl.ds(j * L, L)]
        smax = _bmax(m, idx16)[0]
        cond = smax > tau1

        @pl.when(cond)
        def _():
            sclist[cnt] = j

        return cnt + jnp.where(cond, 1, 0)

    nsc2 = lax.fori_loop(0, nsc, rf_body, jnp.int32(0))

    def newton_it(t, carry):
        tau, changed = carry
        return _newton_step(buf, sclist, nsc2, tau, changed, idx16, zv, onev)

    tau, _ = lax.fori_loop(0, MAX_NEWTON - 1, newton_it, (tau1, ch1))
    return tau, nsc2


def _write_out(buf, outbuf, sclist, nsc2, tau, zv):
    tvo = jnp.full((L,), tau, jnp.float32)

    def wb(u, _):
        base = sclist[u] * SC_ELEMS
        for c in range(CPS):
            v = buf[pl.ds(base + c * L, L)]
            outbuf[pl.ds(base + c * L, L)] = jnp.maximum(v - tvo, zv)
        return 0

    lax.fori_loop(0, nsc2, wb, 0)


def _zero_sup(outbuf, sclist, nsc2, zv):
    def zb(u, _):
        base = sclist[u] * SC_ELEMS
        for c in range(CPS):
            outbuf[pl.ds(base + c * L, L)] = zv
        return 0

    lax.fori_loop(0, nsc2, zb, 0)


def _sc_body(x_hbm, out_hbm, b0, b1, outbuf, segmax, slA, slB,
             si0, si1, si2, si3, so0, so1, so2, so3):
    wid = lax.axis_index("s") * NC + lax.axis_index("c")
    idx16 = lax.iota(jnp.int32, L)
    zv = jnp.zeros((L,), jnp.float32)
    r0 = wid * RPW

    bufs = (b0, b1)
    lists = (slA, slB)
    isems = (si0, si1, si2, si3)
    osems = (so0, so1, so2, so3)

    h_in = [None] * RPW
    h_out = [None] * RPW

    h_in[0] = pltpu.async_copy(x_hbm.at[r0 + 0], b0, isems[0])

    def zb0(i, _):
        outbuf[pl.ds(i * L, L)] = zv
        return 0

    lax.fori_loop(0, N // L, zb0, 0, unroll=16)

    h_in[1] = pltpu.async_copy(x_hbm.at[r0 + 1], b1, isems[1])

    prev_nsc2 = None
    for r in range(RPW):
        h_in[r].wait()
        tau, nsc2 = _compute_row(bufs[r % 2], segmax, lists[r % 2], idx16)
        if r >= 1:
            h_out[r - 1].wait()
            _zero_sup(outbuf, lists[(r - 1) % 2], prev_nsc2, zv)
        _write_out(bufs[r % 2], outbuf, lists[r % 2], nsc2, tau, zv)
        if r + 2 < RPW:
            h_in[r + 2] = pltpu.async_copy(
                x_hbm.at[r0 + r + 2], bufs[r % 2], isems[r + 2])
        h_out[r] = pltpu.async_copy(outbuf, out_hbm.at[r0 + r], osems[r])
        prev_nsc2 = nsc2
    h_out[RPW - 1].wait()


def kernel(input):
    mesh = plsc.VectorSubcoreMesh(
        core_axis_name="c", subcore_axis_name="s", num_cores=NC, num_subcores=NS)
    fn = pl.kernel(
        _sc_body,
        out_type=jax.ShapeDtypeStruct((ROWS, N), jnp.float32),
        mesh=mesh,
        scratch_types=[
            pltpu.VMEM((N,), jnp.float32),
            pltpu.VMEM((N,), jnp.float32),
            pltpu.VMEM((N,), jnp.float32),
            pltpu.VMEM((NSUP * L,), jnp.float32),
            pltpu.SMEM((NSUP,), jnp.int32),
            pltpu.SMEM((NSUP,), jnp.int32),
            pltpu.SemaphoreType.DMA,
            pltpu.SemaphoreType.DMA,
            pltpu.SemaphoreType.DMA,
            pltpu.SemaphoreType.DMA,
            pltpu.SemaphoreType.DMA,
            pltpu.SemaphoreType.DMA,
            pltpu.SemaphoreType.DMA,
            pltpu.SemaphoreType.DMA,
        ],
    )
    return fn(input)
